# baseline (speedup 1.0000x reference)
import functools

import jax
import jax.numpy as jnp
from jax import lax
from jax.experimental import pallas as pl
from jax.experimental.pallas import tpu as pltpu
from jax.experimental.pallas import tpu_sc as plsc

F32 = jnp.float32
I32 = jnp.int32

NC = 2
NS = 16
L = 16
CH = 128
ZR = 56


def _mesh():
    return plsc.VectorSubcoreMesh(core_axis_name="c", subcore_axis_name="s")


def _deg_body(npad, et, e_real, tlz, ei_hbm, ew_hbm, out_hbm, srcp_hbm,
              dstp_hbm, part, dstb, ewb, zbi):
    c = lax.axis_index("c")
    s = lax.axis_index("s")
    w = c * NS + s
    nw = NC * NS
    z16 = jnp.zeros((L,), F32)
    zi16 = jnp.zeros((L,), I32)
    r31 = e_real - (nw - 1) * et

    def zero(i, _):
        part[pl.ds(i * L, L)] = z16
        return 0

    lax.fori_loop(0, npad // L, zero, 0)

    def zeroi(i, _):
        zbi[pl.ds(i * L, L)] = zi16
        return 0

    lax.fori_loop(0, tlz // L, zeroi, 0)

    base0 = w * et
    src_in = ei_hbm.at[0]
    dst_in = ei_hbm.at[1]

    @pl.when(w < nw - 1)
    def _():
        pltpu.sync_copy(dst_in.at[pl.ds(base0, et)], dstb)
        pltpu.sync_copy(ew_hbm.at[pl.ds(base0, et)], ewb)
        pltpu.sync_copy(src_in.at[pl.ds(base0, et)],
                        srcp_hbm.at[pl.ds(base0, et)])
        pltpu.sync_copy(dst_in.at[pl.ds(base0, et)],
                        dstp_hbm.at[pl.ds(base0, et)])

    @pl.when(w == nw - 1)
    def _():
        pltpu.sync_copy(dst_in.at[pl.ds(base0, r31)], dstb.at[pl.ds(0, r31)])
        pltpu.sync_copy(ew_hbm.at[pl.ds(base0, r31)], ewb.at[pl.ds(0, r31)])
        pltpu.sync_copy(src_in.at[pl.ds(base0, r31)],
                        srcp_hbm.at[pl.ds(base0, r31)])
        pltpu.sync_copy(dst_in.at[pl.ds(base0, r31)],
                        dstp_hbm.at[pl.ds(base0, r31)])
        pltpu.sync_copy(zbi, srcp_hbm.at[pl.ds(e_real, tlz)])
        pltpu.sync_copy(zbi, dstp_hbm.at[pl.ds(e_real, tlz)])

        def ztail(i, _):
            dstb[pl.ds(r31 + i * L, L)] = zi16
            ewb[pl.ds(r31 + i * L, L)] = z16
            return 0

        lax.fori_loop(0, (et - r31) // L, ztail, 0)

    def group(g, _):
        d16 = dstb[pl.ds(g * L, L)]
        e16 = ewb[pl.ds(g * L, L)]
        plsc.addupdate_scatter(part, [d16], e16)
        return 0

    lax.fori_loop(0, et // L, group, 0)
    pltpu.sync_copy(part, out_hbm.at[w])


def _make_deg(npad, epad, e_real):
    et = epad // (NC * NS)
    tlz = epad + SB - e_real
    return pl.kernel(
        functools.partial(_deg_body, npad, et, e_real, tlz),
        out_type=[
            jax.ShapeDtypeStruct((NC * NS, npad), F32),
            jax.ShapeDtypeStruct((epad + SB,), I32),
            jax.ShapeDtypeStruct((epad + SB,), I32),
        ],
        mesh=_mesh(),
        compiler_params=pltpu.CompilerParams(needs_layout_passes=False),
        scratch_types=[
            pltpu.VMEM((npad,), F32),
            pltpu.VMEM((et,), I32),
            pltpu.VMEM((et,), F32),
            pltpu.VMEM((tlz,), I32),
        ],
    )


SB = 512
CPS = SB // CH


def _agg_body(npad, et, d, feature_split, out_cols, src_hbm, dst_hbm, ew_hbm,
              v_hbm, out_hbm, srcb, dstb, ewb, idxg, idxs, rows, zbuf, acc,
              sg0, sg1, sg2, sg3, ss0, ss1, ss2, ss3):
    c = lax.axis_index("c")
    s = lax.axis_index("s")
    tr = npad // NS
    nvec = d // L
    n_super = et // SB
    z16 = jnp.zeros((L,), F32)
    zi16 = jnp.zeros((L,), I32)
    sg = (sg0, sg1, sg2, sg3)
    ss = (ss0, ss1, ss2, ss3)

    def zb(i, _):
        for k in range(nvec):
            zbuf[i, pl.ds(k * L, L)] = z16
        return 0

    lax.fori_loop(0, ZR, zb, 0)

    row0 = s * tr

    def za(j, _):
        pltpu.sync_copy(zbuf, acc.at[pl.ds(row0 + j * ZR, ZR)])
        return 0

    lax.fori_loop(0, tr // ZR, za, 0)
    plsc.subcore_barrier()

    if feature_split:
        tile_base = s * et
        goff = c * npad
    else:
        tile_base = (c * NS + s) * et
        goff = 0

    def load_edges(sc_i):
        eo = lax.rem(sc_i, 2) * SB
        b = tile_base + sc_i * SB
        pltpu.sync_copy(src_hbm.at[pl.ds(b, SB)], srcb.at[pl.ds(eo, SB)])
        pltpu.sync_copy(dst_hbm.at[pl.ds(b, SB)], dstb.at[pl.ds(eo, SB)])
        pltpu.sync_copy(ew_hbm.at[pl.ds(b, SB)], ewb.at[pl.ds(eo, SB)])

    def build_idx(p, off):
        for g in range(CH // L):
            sl = pl.ds(off + g * L, L)
            idxg[p, pl.ds(g * L, L)] = srcb[sl] + goff
            idxs[p, pl.ds(g * L, L)] = dstb[sl]

    def issue_gather(p):
        pltpu.async_copy(v_hbm.at[idxg.at[p]], rows.at[p], sg[p])

    def wait_gather(p):
        pltpu.make_async_copy(v_hbm.at[idxg.at[p]], rows.at[p], sg[p]).wait()

    def issue_scatter(p):
        pltpu.async_copy(rows.at[p], acc.at[idxs.at[p]], ss[p], add=True)

    def wait_scatter(p):
        pltpu.make_async_copy(rows.at[p], acc.at[idxs.at[p]], ss[p]).wait()

    def scale(p, off):
        def body(g, _):
            e16 = ewb[pl.ds(off + g * L, L)]
            for e in range(L):
                sc = e16[e]
                for k in range(nvec):
                    rows[p, g * L + e, pl.ds(k * L, L)] = (
                        rows[p, g * L + e, pl.ds(k * L, L)] * sc)
            return 0

        lax.fori_loop(0, CH // L, body, 0)

    for q in (2, 3):
        for g in range(CH // L):
            idxs[q, pl.ds(g * L, L)] = zi16

        def zr(i, _):
            for k in range(nvec):
                rows[q, i, pl.ds(k * L, L)] = z16
            return 0

        lax.fori_loop(0, CH, zr, 0)
        issue_scatter(q)

    load_edges(0)
    build_idx(0, 0)
    issue_gather(0)
    build_idx(1, CH)
    issue_gather(1)

    def superchunk(sc_i, _):
        eo_cur = lax.rem(sc_i, 2) * SB
        eo_next = SB - eo_cur
        load_edges(sc_i + 1)
        for j in range(CPS):
            p = j
            q = (j + 2) % 4
            wait_gather(p)
            wait_scatter(q)
            if j < 2:
                build_idx(q, eo_cur + (j + 2) * CH)
            else:
                build_idx(q, eo_next + (j - 2) * CH)
            issue_gather(q)
            scale(p, eo_cur + j * CH)
            issue_scatter(p)
        return 0

    lax.fori_loop(0, n_super, superchunk, 0)
    wait_gather(0)
    wait_gather(1)
    wait_scatter(2)
    wait_scatter(3)

    plsc.subcore_barrier()
    if out_cols == d:
        pltpu.sync_copy(acc.at[pl.ds(row0, tr)],
                        out_hbm.at[pl.ds(c * npad + row0, tr)])
    else:
        pltpu.sync_copy(acc.at[pl.ds(row0, tr), pl.ds(0, out_cols)],
                        out_hbm.at[pl.ds(c * npad + row0, tr)])


def _make_agg(npad, epad, d, feature_split, out_cols):
    et = epad // ((NS if feature_split else NC * NS))
    return pl.kernel(
        functools.partial(_agg_body, npad, et, d, feature_split, out_cols),
        out_type=jax.ShapeDtypeStruct((NC * npad, out_cols), F32),
        mesh=_mesh(),
        compiler_params=pltpu.CompilerParams(
            needs_layout_passes=False, use_tc_tiling_on_sc=False),
        scratch_types=[
            pltpu.VMEM((2 * SB,), I32),
            pltpu.VMEM((2 * SB,), I32),
            pltpu.VMEM((2 * SB,), F32),
            pltpu.VMEM((4, CH), I32),
            pltpu.VMEM((4, CH), I32),
            pltpu.VMEM((4, CH, d), F32),
            pltpu.VMEM((ZR, d), F32),
            pltpu.VMEM_SHARED((npad, d), F32),
            pltpu.SemaphoreType.DMA,
            pltpu.SemaphoreType.DMA,
            pltpu.SemaphoreType.DMA,
            pltpu.SemaphoreType.DMA,
            pltpu.SemaphoreType.DMA,
            pltpu.SemaphoreType.DMA,
            pltpu.SemaphoreType.DMA,
            pltpu.SemaphoreType.DMA,
        ],
    )


def _hid_body(npad, b4, s1_hbm, xs_hbm, dis_hbm, b1_hbm, w2_hbm,
              hs2p_hbm, hs2c_hbm, s0b, s1b, x0b, x1b, disb, b1b, w2b,
              hpb, hcb, smh):
    c = lax.axis_index("c")
    s = lax.axis_index("s")
    w = c * NS + s
    nt = npad // (NC * NS)
    iota = lax.iota(I32, L)
    z16 = jnp.zeros((L,), F32)
    zi16 = jnp.zeros((L,), I32)
    oi16 = jnp.full((L,), 1, I32)

    pltpu.sync_copy(b1_hbm, b1b)
    pltpu.sync_copy(w2_hbm, w2b)

    def zhp(i, _):
        hpb[i, pl.ds(0, L)] = z16
        return 0

    lax.fori_loop(0, b4, zhp, 0)

    base = w * nt

    def blk(bi, _):
        nb0 = base + bi * b4
        d1 = pltpu.async_copy(s1_hbm.at[pl.ds(nb0, b4)],
                              s0b.at[:, pl.ds(0, 32)], smh)
        d2 = pltpu.async_copy(s1_hbm.at[pl.ds(npad + nb0, b4)],
                              s1b.at[:, pl.ds(0, 32)], smh)
        d3 = pltpu.async_copy(xs_hbm.at[pl.ds(nb0, b4)],
                              x0b.at[:, pl.ds(0, 32)], smh)
        d4 = pltpu.async_copy(xs_hbm.at[pl.ds(npad + nb0, b4)],
                              x1b.at[:, pl.ds(0, 32)], smh)
        d5 = pltpu.async_copy(dis_hbm.at[pl.ds(0, 1), pl.ds(nb0, b4)],
                              disb, smh)
        d1.wait()
        d2.wait()
        d3.wait()
        d4.wait()
        d5.wait()

        def grp(g, _):
            rows16 = g * L + iota
            dis16 = disb[0, pl.ds(g * L, L)]
            b1vs = [b1b[pl.ds(q * L, L)] for q in range(4)]
            w2vs = [w2b[pl.ds(q * L, L)] for q in range(8)]
            acc0 = z16
            acc1 = z16
            for k in range(4 * L):
                sb = s0b if k < 2 * L else s1b
                xb = x0b if k < 2 * L else x1b
                colv = jnp.full((L,), k % (2 * L), I32)
                sv = plsc.load_gather(sb, [rows16, colv])
                xv = plsc.load_gather(xb, [rows16, colv])
                h = jnp.maximum(dis16 * (sv + xv) + b1vs[k // L][k % L], 0.0)
                acc0 = acc0 + h * w2vs[(2 * k) // L][(2 * k) % L]
                acc1 = acc1 + h * w2vs[(2 * k + 1) // L][(2 * k + 1) % L]
            o0 = dis16 * acc0
            o1 = dis16 * acc1
            plsc.store_scatter(hpb, [rows16, zi16], o0)
            plsc.store_scatter(hpb, [rows16, oi16], o1)
            plsc.store_scatter(hcb, [rows16, zi16], o0)
            plsc.store_scatter(hcb, [rows16, oi16], o1)
            return 0

        lax.fori_loop(0, b4 // L, grp, 0)
        pltpu.sync_copy(hpb.at[:, pl.ds(0, 16)], hs2p_hbm.at[pl.ds(nb0, b4)])
        pltpu.sync_copy(hcb, hs2c_hbm.at[pl.ds(nb0, b4)])
        return 0

    lax.fori_loop(0, nt // b4, blk, 0)


def _make_hid(npad, d2p, d_out):
    b4 = 224
    return pl.kernel(
        functools.partial(_hid_body, npad, b4),
        out_type=[
            jax.ShapeDtypeStruct((npad, d2p), F32),
            jax.ShapeDtypeStruct((npad, d_out), F32),
        ],
        mesh=_mesh(),
        compiler_params=pltpu.CompilerParams(
            needs_layout_passes=False, use_tc_tiling_on_sc=False),
        scratch_types=[
            pltpu.VMEM((b4, 33), F32),
            pltpu.VMEM((b4, 33), F32),
            pltpu.VMEM((b4, 33), F32),
            pltpu.VMEM((b4, 33), F32),
            pltpu.VMEM((1, b4), F32),
            pltpu.VMEM((64,), F32),
            pltpu.VMEM((128,), F32),
            pltpu.VMEM((b4, 17), F32),
            pltpu.VMEM((b4, d_out), F32),
            pltpu.SemaphoreType.DMA,
        ],
    )


_LOG2_C = (0.04343132, -0.40488876, 1.59397599, -3.4926196, 5.04697861,
           -2.78684575)
_LN2 = 0.6931471805599453


def _out_body(npad, n, b4, dis_hbm, s2_hbm, hc_hbm, b2_hbm, out_hbm,
              s2a, s2b, hcb, disb, b2b, ob, sem):
    c = lax.axis_index("c")
    s = lax.axis_index("s")
    w = c * NS + s
    nt = npad // (NC * NS)
    iota = lax.iota(I32, L)
    zi16 = jnp.zeros((L,), I32)
    oi16 = jnp.full((L,), 1, I32)

    pltpu.sync_copy(b2_hbm, b2b)
    base = w * nt

    def blk(bi, _):
        nb0 = base + bi * b4
        d1 = pltpu.async_copy(s2_hbm.at[pl.ds(nb0, b4)],
                              s2a.at[:, pl.ds(0, 16)], sem)
        d2 = pltpu.async_copy(s2_hbm.at[pl.ds(npad + nb0, b4)],
                              s2b.at[:, pl.ds(0, 16)], sem)
        d3 = pltpu.async_copy(hc_hbm.at[pl.ds(nb0, b4)], hcb, sem)
        d4 = pltpu.async_copy(dis_hbm.at[pl.ds(0, 1), pl.ds(nb0, b4)],
                              disb, sem)
        d1.wait()
        d2.wait()
        d3.wait()
        d4.wait()
        b2v = b2b[pl.ds(0, L)]

        def grp(g, _):
            rows16 = g * L + iota
            dis16 = disb[0, pl.ds(g * L, L)]
            a0 = plsc.load_gather(s2a, [rows16, zi16])
            a1 = plsc.load_gather(s2a, [rows16, oi16])
            c0 = plsc.load_gather(s2b, [rows16, zi16])
            c1 = plsc.load_gather(s2b, [rows16, oi16])
            h0 = plsc.load_gather(hcb, [rows16, zi16])
            h1 = plsc.load_gather(hcb, [rows16, oi16])
            o0 = dis16 * (a0 + c0 + h0) + b2v[0]
            o1 = dis16 * (a1 + c1 + h1) + b2v[1]
            mx = jnp.maximum(o0, o1)
            t = jnp.exp(-jnp.abs(o0 - o1))
            y = 1.0 + t
            bits = plsc.bitcast(y, jnp.int32)
            ev = (lax.shift_right_logical(bits, 23) & 255) - 127
            mant = plsc.bitcast(
                (bits & 0x7FFFFF) | 0x3F800000, F32)
            p = jnp.full((L,), _LOG2_C[0], F32)
            for cc in _LOG2_C[1:]:
                p = p * mant + cc
            lse = mx + _LN2 * (p + ev.astype(F32))
            plsc.store_scatter(ob, [rows16, zi16], o0 - lse)
            plsc.store_scatter(ob, [rows16, oi16], o1 - lse)
            return 0

        lax.fori_loop(0, b4 // L, grp, 0)
        @pl.when(nb0 + b4 <= n)
        def _():
            pltpu.sync_copy(ob, out_hbm.at[pl.ds(nb0, b4)])

        return 0

    lax.fori_loop(0, nt // b4, blk, 0)

    tail_start = (n // b4) * b4
    tail = n - tail_start
    if tail:
        tw = tail_start // nt
        @pl.when(w == tw)
        def _():
            pltpu.sync_copy(ob.at[pl.ds(0, tail)],
                            out_hbm.at[pl.ds(tail_start, tail)])


def _make_out(npad, n, d_out):
    b4 = 224
    return pl.kernel(
        functools.partial(_out_body, npad, n, b4),
        out_type=jax.ShapeDtypeStruct((n, d_out), F32),
        mesh=_mesh(),
        compiler_params=pltpu.CompilerParams(
            needs_layout_passes=False, use_tc_tiling_on_sc=False),
        scratch_types=[
            pltpu.VMEM((b4, 17), F32),
            pltpu.VMEM((b4, 17), F32),
            pltpu.VMEM((b4, d_out), F32),
            pltpu.VMEM((1, b4), F32),
            pltpu.VMEM((L,), F32),
            pltpu.VMEM((b4, d_out), F32),
            pltpu.SemaphoreType.DMA,
        ],
    )


def _k2_body(parts_ref, x_ref, w1_ref, xs_ref, dis1d_ref):
    parts = parts_ref[...]
    nparts = parts.shape[0]
    ones = jnp.ones((nparts, 1), F32)
    deg = 1.0 + lax.dot_general(parts, ones, (((0,), (0,)), ((), ())),
                                preferred_element_type=F32)
    dis = lax.rsqrt(deg)
    deg_row = 1.0 + jnp.dot(jnp.ones((1, nparts), F32), parts,
                            preferred_element_type=F32)
    dis1d_ref[...] = lax.rsqrt(deg_row)
    xw = jnp.dot(x_ref[...], w1_ref[...], preferred_element_type=F32,
                 precision=lax.Precision.HIGHEST)
    half = xw.shape[1] // 2
    xs_ref[0] = dis * xw[:, :half]
    xs_ref[1] = dis * xw[:, half:]


def kernel(x, edge_index, edge_weight, W1, b1, W2, b2):
    n, d_in = x.shape
    e = edge_index.shape[1]
    d_hid = W1.shape[1]
    d_out = W2.shape[1]
    half = d_hid // 2
    d2p = 16

    npad = -(-n // 512) * 512
    r = 3584 if npad % 3584 == 0 else 512
    nb = npad // r
    epad = -(-e // (NC * NS * CH)) * (NC * NS * CH)

    ew = jnp.pad(edge_weight, (0, epad + SB - e))
    xp = x

    parts, src, dst = _make_deg(npad, epad, e)(edge_index, edge_weight)

    xs3, dis1d = pl.pallas_call(
        _k2_body,
        grid=(nb,),
        in_specs=[
            pl.BlockSpec((NC * NS, r), lambda j: (0, j)),
            pl.BlockSpec((r, d_in), lambda j: (j, 0)),
            pl.BlockSpec((d_in, d_hid), lambda j: (0, 0)),
        ],
        out_specs=[
            pl.BlockSpec((NC, r, half), lambda j: (0, j, 0)),
            pl.BlockSpec((1, r), lambda j: (0, j)),
        ],
        out_shape=[
            jax.ShapeDtypeStruct((NC, npad, half), F32),
            jax.ShapeDtypeStruct((1, npad), F32),
        ],
    )(parts, xp, W1)
    xs_flat = xs3.reshape(NC * npad, half)

    s1_flat = _make_agg(npad, epad, half, True, half)(src, dst, ew, xs_flat)

    hs2p, hs2c = _make_hid(npad, d2p, d_out)(
        s1_flat, xs_flat, dis1d, b1, W2.reshape(d_hid * d_out))

    s2c = _make_agg(npad, epad, d2p, False, d2p)(src, dst, ew, hs2p)

    return _make_out(npad, n, d_out)(
        dis1d, s2c, hs2c, jnp.pad(b2, (0, L - d_out)))


# speedup vs baseline: 1.1920x; 1.1920x over previous
import functools

import jax
import jax.numpy as jnp
from jax import lax
from jax.experimental import pallas as pl
from jax.experimental.pallas import tpu as pltpu
from jax.experimental.pallas import tpu_sc as plsc

F32 = jnp.float32
I32 = jnp.int32

NC = 2
NS = 16
L = 16
CH = 128
ZR = 56


def _mesh():
    return plsc.VectorSubcoreMesh(core_axis_name="c", subcore_axis_name="s")


def _deg_body(npad, et, dst_hbm, ew_hbm, out_hbm, part, dstb, ewb):
    c = lax.axis_index("c")
    s = lax.axis_index("s")
    w = c * NS + s
    z16 = jnp.zeros((L,), F32)

    def zero(i, _):
        part[pl.ds(i * L, L)] = z16
        return 0

    lax.fori_loop(0, npad // L, zero, 0)

    base0 = w * et
    pltpu.sync_copy(dst_hbm.at[pl.ds(base0, et)], dstb)
    pltpu.sync_copy(ew_hbm.at[pl.ds(base0, et)], ewb)

    def group(g, _):
        d16 = dstb[pl.ds(g * L, L)]
        e16 = ewb[pl.ds(g * L, L)]
        plsc.addupdate_scatter(part, [d16], e16)
        return 0

    lax.fori_loop(0, et // L, group, 0)
    pltpu.sync_copy(part, out_hbm.at[w])


def _make_deg(npad, epad):
    et = epad // (NC * NS)
    return pl.kernel(
        functools.partial(_deg_body, npad, et),
        out_type=jax.ShapeDtypeStruct((NC * NS, npad), F32),
        mesh=_mesh(),
        compiler_params=pltpu.CompilerParams(needs_layout_passes=False),
        scratch_types=[
            pltpu.VMEM((npad,), F32),
            pltpu.VMEM((et,), I32),
            pltpu.VMEM((et,), F32),
        ],
    )


SB = 512
CPS = SB // CH


def _agg_body(npad, et, d, feature_split, out_cols, src_hbm, dst_hbm, ew_hbm,
              v_hbm, out_hbm, srcb, dstb, ewb, idxg, idxs, rows, zbuf, acc,
              sg0, sg1, sg2, sg3, ss0, ss1, ss2, ss3):
    c = lax.axis_index("c")
    s = lax.axis_index("s")
    tr = npad // NS
    nvec = d // L
    n_super = et // SB
    z16 = jnp.zeros((L,), F32)
    zi16 = jnp.zeros((L,), I32)
    sg = (sg0, sg1, sg2, sg3)
    ss = (ss0, ss1, ss2, ss3)

    def zb(i, _):
        for k in range(nvec):
            zbuf[i, pl.ds(k * L, L)] = z16
        return 0

    lax.fori_loop(0, ZR, zb, 0)

    row0 = s * tr

    def za(j, _):
        pltpu.sync_copy(zbuf, acc.at[pl.ds(row0 + j * ZR, ZR)])
        return 0

    lax.fori_loop(0, tr // ZR, za, 0)
    plsc.subcore_barrier()

    if feature_split:
        tile_base = s * et
        goff = c * npad
    else:
        tile_base = (c * NS + s) * et
        goff = 0

    def load_edges(sc_i):
        eo = lax.rem(sc_i, 2) * SB
        b = tile_base + sc_i * SB
        pltpu.sync_copy(src_hbm.at[pl.ds(b, SB)], srcb.at[pl.ds(eo, SB)])
        pltpu.sync_copy(dst_hbm.at[pl.ds(b, SB)], dstb.at[pl.ds(eo, SB)])
        pltpu.sync_copy(ew_hbm.at[pl.ds(b, SB)], ewb.at[pl.ds(eo, SB)])

    def build_idx(p, off):
        for g in range(CH // L):
            sl = pl.ds(off + g * L, L)
            idxg[p, pl.ds(g * L, L)] = srcb[sl] + goff
            idxs[p, pl.ds(g * L, L)] = dstb[sl]

    def issue_gather(p):
        pltpu.async_copy(v_hbm.at[idxg.at[p]], rows.at[p], sg[p])

    def wait_gather(p):
        pltpu.make_async_copy(v_hbm.at[idxg.at[p]], rows.at[p], sg[p]).wait()

    def issue_scatter(p):
        pltpu.async_copy(rows.at[p], acc.at[idxs.at[p]], ss[p], add=True)

    def wait_scatter(p):
        pltpu.make_async_copy(rows.at[p], acc.at[idxs.at[p]], ss[p]).wait()

    def scale(p, off):
        def body(g, _):
            e16 = ewb[pl.ds(off + g * L, L)]
            for e in range(L):
                sc = e16[e]
                for k in range(nvec):
                    rows[p, g * L + e, pl.ds(k * L, L)] = (
                        rows[p, g * L + e, pl.ds(k * L, L)] * sc)
            return 0

        lax.fori_loop(0, CH // L, body, 0)

    for q in (2, 3):
        for g in range(CH // L):
            idxs[q, pl.ds(g * L, L)] = zi16

        def zr(i, _):
            for k in range(nvec):
                rows[q, i, pl.ds(k * L, L)] = z16
            return 0

        lax.fori_loop(0, CH, zr, 0)
        issue_scatter(q)

    load_edges(0)
    build_idx(0, 0)
    issue_gather(0)
    build_idx(1, CH)
    issue_gather(1)

    def superchunk(sc_i, _):
        eo_cur = lax.rem(sc_i, 2) * SB
        eo_next = SB - eo_cur
        load_edges(sc_i + 1)
        for j in range(CPS):
            p = j
            q = (j + 2) % 4
            wait_gather(p)
            wait_scatter(q)
            if j < 2:
                build_idx(q, eo_cur + (j + 2) * CH)
            else:
                build_idx(q, eo_next + (j - 2) * CH)
            issue_gather(q)
            scale(p, eo_cur + j * CH)
            issue_scatter(p)
        return 0

    lax.fori_loop(0, n_super, superchunk, 0)
    wait_gather(0)
    wait_gather(1)
    wait_scatter(2)
    wait_scatter(3)

    plsc.subcore_barrier()
    if out_cols == d:
        pltpu.sync_copy(acc.at[pl.ds(row0, tr)],
                        out_hbm.at[pl.ds(c * npad + row0, tr)])
    else:
        pltpu.sync_copy(acc.at[pl.ds(row0, tr), pl.ds(0, out_cols)],
                        out_hbm.at[pl.ds(c * npad + row0, tr)])


def _make_agg(npad, epad, d, feature_split, out_cols):
    et = epad // ((NS if feature_split else NC * NS))
    return pl.kernel(
        functools.partial(_agg_body, npad, et, d, feature_split, out_cols),
        out_type=jax.ShapeDtypeStruct((NC * npad, out_cols), F32),
        mesh=_mesh(),
        compiler_params=pltpu.CompilerParams(
            needs_layout_passes=False, use_tc_tiling_on_sc=False),
        scratch_types=[
            pltpu.VMEM((2 * SB,), I32),
            pltpu.VMEM((2 * SB,), I32),
            pltpu.VMEM((2 * SB,), F32),
            pltpu.VMEM((4, CH), I32),
            pltpu.VMEM((4, CH), I32),
            pltpu.VMEM((4, CH, d), F32),
            pltpu.VMEM((ZR, d), F32),
            pltpu.VMEM_SHARED((npad, d), F32),
            pltpu.SemaphoreType.DMA,
            pltpu.SemaphoreType.DMA,
            pltpu.SemaphoreType.DMA,
            pltpu.SemaphoreType.DMA,
            pltpu.SemaphoreType.DMA,
            pltpu.SemaphoreType.DMA,
            pltpu.SemaphoreType.DMA,
            pltpu.SemaphoreType.DMA,
        ],
    )


def _hid_body(npad, b4, s1_hbm, xs_hbm, dis_hbm, b1_hbm, w2_hbm,
              hs2p_hbm, hs2c_hbm, s0b, s1b, x0b, x1b, disb, b1b, w2b,
              hpb, hcb, smh):
    c = lax.axis_index("c")
    s = lax.axis_index("s")
    w = c * NS + s
    nt = npad // (NC * NS)
    iota = lax.iota(I32, L)
    z16 = jnp.zeros((L,), F32)
    zi16 = jnp.zeros((L,), I32)
    oi16 = jnp.full((L,), 1, I32)

    pltpu.sync_copy(b1_hbm, b1b)
    pltpu.sync_copy(w2_hbm, w2b)

    def zhp(i, _):
        hpb[i, pl.ds(0, L)] = z16
        return 0

    lax.fori_loop(0, b4, zhp, 0)

    base = w * nt

    def blk(bi, _):
        nb0 = base + bi * b4
        d1 = pltpu.async_copy(s1_hbm.at[pl.ds(nb0, b4)],
                              s0b.at[:, pl.ds(0, 32)], smh)
        d2 = pltpu.async_copy(s1_hbm.at[pl.ds(npad + nb0, b4)],
                              s1b.at[:, pl.ds(0, 32)], smh)
        d3 = pltpu.async_copy(xs_hbm.at[pl.ds(nb0, b4)],
                              x0b.at[:, pl.ds(0, 32)], smh)
        d4 = pltpu.async_copy(xs_hbm.at[pl.ds(npad + nb0, b4)],
                              x1b.at[:, pl.ds(0, 32)], smh)
        d5 = pltpu.async_copy(dis_hbm.at[pl.ds(0, 1), pl.ds(nb0, b4)],
                              disb, smh)
        d1.wait()
        d2.wait()
        d3.wait()
        d4.wait()
        d5.wait()

        def grp(g, _):
            rows16 = g * L + iota
            dis16 = disb[0, pl.ds(g * L, L)]
            b1vs = [b1b[pl.ds(q * L, L)] for q in range(4)]
            w2vs = [w2b[pl.ds(q * L, L)] for q in range(8)]
            acc0 = z16
            acc1 = z16
            for k in range(4 * L):
                sb = s0b if k < 2 * L else s1b
                xb = x0b if k < 2 * L else x1b
                colv = jnp.full((L,), k % (2 * L), I32)
                sv = plsc.load_gather(sb, [rows16, colv])
                xv = plsc.load_gather(xb, [rows16, colv])
                h = jnp.maximum(dis16 * (sv + xv) + b1vs[k // L][k % L], 0.0)
                acc0 = acc0 + h * w2vs[(2 * k) // L][(2 * k) % L]
                acc1 = acc1 + h * w2vs[(2 * k + 1) // L][(2 * k + 1) % L]
            o0 = dis16 * acc0
            o1 = dis16 * acc1
            plsc.store_scatter(hpb, [rows16, zi16], o0)
            plsc.store_scatter(hpb, [rows16, oi16], o1)
            plsc.store_scatter(hcb, [rows16, zi16], o0)
            plsc.store_scatter(hcb, [rows16, oi16], o1)
            return 0

        lax.fori_loop(0, b4 // L, grp, 0)
        pltpu.sync_copy(hpb.at[:, pl.ds(0, 16)], hs2p_hbm.at[pl.ds(nb0, b4)])
        pltpu.sync_copy(hcb, hs2c_hbm.at[pl.ds(nb0, b4)])
        return 0

    lax.fori_loop(0, nt // b4, blk, 0)


def _make_hid(npad, d2p, d_out):
    b4 = 224
    return pl.kernel(
        functools.partial(_hid_body, npad, b4),
        out_type=[
            jax.ShapeDtypeStruct((npad, d2p), F32),
            jax.ShapeDtypeStruct((npad, d_out), F32),
        ],
        mesh=_mesh(),
        compiler_params=pltpu.CompilerParams(
            needs_layout_passes=False, use_tc_tiling_on_sc=False),
        scratch_types=[
            pltpu.VMEM((b4, 33), F32),
            pltpu.VMEM((b4, 33), F32),
            pltpu.VMEM((b4, 33), F32),
            pltpu.VMEM((b4, 33), F32),
            pltpu.VMEM((1, b4), F32),
            pltpu.VMEM((64,), F32),
            pltpu.VMEM((128,), F32),
            pltpu.VMEM((b4, 17), F32),
            pltpu.VMEM((b4, d_out), F32),
            pltpu.SemaphoreType.DMA,
        ],
    )


_LOG2_C = (0.04343132, -0.40488876, 1.59397599, -3.4926196, 5.04697861,
           -2.78684575)
_LN2 = 0.6931471805599453


def _out_body(npad, n, b4, dis_hbm, s2_hbm, hc_hbm, b2_hbm, out_hbm,
              s2a, s2b, hcb, disb, b2b, ob, sem):
    c = lax.axis_index("c")
    s = lax.axis_index("s")
    w = c * NS + s
    nt = npad // (NC * NS)
    iota = lax.iota(I32, L)
    zi16 = jnp.zeros((L,), I32)
    oi16 = jnp.full((L,), 1, I32)

    pltpu.sync_copy(b2_hbm, b2b)
    base = w * nt

    def blk(bi, _):
        nb0 = base + bi * b4
        d1 = pltpu.async_copy(s2_hbm.at[pl.ds(nb0, b4)],
                              s2a.at[:, pl.ds(0, 16)], sem)
        d2 = pltpu.async_copy(s2_hbm.at[pl.ds(npad + nb0, b4)],
                              s2b.at[:, pl.ds(0, 16)], sem)
        d3 = pltpu.async_copy(hc_hbm.at[pl.ds(nb0, b4)], hcb, sem)
        d4 = pltpu.async_copy(dis_hbm.at[pl.ds(0, 1), pl.ds(nb0, b4)],
                              disb, sem)
        d1.wait()
        d2.wait()
        d3.wait()
        d4.wait()
        b2v = b2b[pl.ds(0, L)]

        def grp(g, _):
            rows16 = g * L + iota
            dis16 = disb[0, pl.ds(g * L, L)]
            a0 = plsc.load_gather(s2a, [rows16, zi16])
            a1 = plsc.load_gather(s2a, [rows16, oi16])
            c0 = plsc.load_gather(s2b, [rows16, zi16])
            c1 = plsc.load_gather(s2b, [rows16, oi16])
            h0 = plsc.load_gather(hcb, [rows16, zi16])
            h1 = plsc.load_gather(hcb, [rows16, oi16])
            o0 = dis16 * (a0 + c0 + h0) + b2v[0]
            o1 = dis16 * (a1 + c1 + h1) + b2v[1]
            mx = jnp.maximum(o0, o1)
            t = jnp.exp(-jnp.abs(o0 - o1))
            y = 1.0 + t
            bits = plsc.bitcast(y, jnp.int32)
            ev = (lax.shift_right_logical(bits, 23) & 255) - 127
            mant = plsc.bitcast(
                (bits & 0x7FFFFF) | 0x3F800000, F32)
            p = jnp.full((L,), _LOG2_C[0], F32)
            for cc in _LOG2_C[1:]:
                p = p * mant + cc
            lse = mx + _LN2 * (p + ev.astype(F32))
            plsc.store_scatter(ob, [rows16, zi16], o0 - lse)
            plsc.store_scatter(ob, [rows16, oi16], o1 - lse)
            return 0

        lax.fori_loop(0, b4 // L, grp, 0)
        @pl.when(nb0 + b4 <= n)
        def _():
            pltpu.sync_copy(ob, out_hbm.at[pl.ds(nb0, b4)])

        return 0

    lax.fori_loop(0, nt // b4, blk, 0)

    tail_start = (n // b4) * b4
    tail = n - tail_start
    if tail:
        tw = tail_start // nt
        @pl.when(w == tw)
        def _():
            pltpu.sync_copy(ob.at[pl.ds(0, tail)],
                            out_hbm.at[pl.ds(tail_start, tail)])


def _make_out(npad, n, d_out):
    b4 = 224
    return pl.kernel(
        functools.partial(_out_body, npad, n, b4),
        out_type=jax.ShapeDtypeStruct((n, d_out), F32),
        mesh=_mesh(),
        compiler_params=pltpu.CompilerParams(
            needs_layout_passes=False, use_tc_tiling_on_sc=False),
        scratch_types=[
            pltpu.VMEM((b4, 17), F32),
            pltpu.VMEM((b4, 17), F32),
            pltpu.VMEM((b4, d_out), F32),
            pltpu.VMEM((1, b4), F32),
            pltpu.VMEM((L,), F32),
            pltpu.VMEM((b4, d_out), F32),
            pltpu.SemaphoreType.DMA,
        ],
    )


def _k2_body(parts_ref, x_ref, w1_ref, xs_ref, dis1d_ref):
    parts = parts_ref[...]
    nparts = parts.shape[0]
    ones = jnp.ones((nparts, 1), F32)
    deg = 1.0 + lax.dot_general(parts, ones, (((0,), (0,)), ((), ())),
                                preferred_element_type=F32)
    dis = lax.rsqrt(deg)
    deg_row = 1.0 + jnp.dot(jnp.ones((1, nparts), F32), parts,
                            preferred_element_type=F32)
    dis1d_ref[...] = lax.rsqrt(deg_row)
    xw = jnp.dot(x_ref[...], w1_ref[...], preferred_element_type=F32,
                 precision=lax.Precision.HIGHEST)
    half = xw.shape[1] // 2
    xs_ref[0] = dis * xw[:, :half]
    xs_ref[1] = dis * xw[:, half:]


def kernel(x, edge_index, edge_weight, W1, b1, W2, b2):
    n, d_in = x.shape
    e = edge_index.shape[1]
    d_hid = W1.shape[1]
    d_out = W2.shape[1]
    half = d_hid // 2
    d2p = 16

    npad = -(-n // 512) * 512
    r = 3584 if npad % 3584 == 0 else 512
    nb = npad // r
    epad = -(-e // (NC * NS * CH)) * (NC * NS * CH)

    ei = jnp.pad(edge_index, ((0, 0), (0, epad + SB - e)))
    src = ei[0]
    dst = ei[1]
    ew = jnp.pad(edge_weight, (0, epad + SB - e))
    xp = x

    parts = _make_deg(npad, epad)(dst, ew)

    xs3, dis1d = pl.pallas_call(
        _k2_body,
        grid=(nb,),
        in_specs=[
            pl.BlockSpec((NC * NS, r), lambda j: (0, j)),
            pl.BlockSpec((r, d_in), lambda j: (j, 0)),
            pl.BlockSpec((d_in, d_hid), lambda j: (0, 0)),
        ],
        out_specs=[
            pl.BlockSpec((NC, r, half), lambda j: (0, j, 0)),
            pl.BlockSpec((1, r), lambda j: (0, j)),
        ],
        out_shape=[
            jax.ShapeDtypeStruct((NC, npad, half), F32),
            jax.ShapeDtypeStruct((1, npad), F32),
        ],
    )(parts, xp, W1)
    xs_flat = xs3.reshape(NC * npad, half)

    s1_flat = _make_agg(npad, epad, half, True, half)(src, dst, ew, xs_flat)

    hs2p, hs2c = _make_hid(npad, d2p, d_out)(
        s1_flat, xs_flat, dis1d, b1, W2.reshape(d_hid * d_out))

    s2c = _make_agg(npad, epad, d2p, False, d2p)(src, dst, ew, hs2p)

    return _make_out(npad, n, d_out)(
        dis1d, s2c, hs2c, jnp.pad(b2, (0, L - d_out)))


# speedup vs baseline: 1.1991x; 1.0060x over previous
import functools

import jax
import jax.numpy as jnp
from jax import lax
from jax.experimental import pallas as pl
from jax.experimental.pallas import tpu as pltpu
from jax.experimental.pallas import tpu_sc as plsc

F32 = jnp.float32
I32 = jnp.int32

NC = 2
NS = 16
L = 16
CH = 128
ZR = 56


def _mesh():
    return plsc.VectorSubcoreMesh(core_axis_name="c", subcore_axis_name="s")


def _deg_body(npad, et, dst_hbm, ew_hbm, out_hbm, part, dstb, ewb):
    c = lax.axis_index("c")
    s = lax.axis_index("s")
    w = c * NS + s
    z16 = jnp.zeros((L,), F32)

    def zero(i, _):
        part[pl.ds(i * L, L)] = z16
        return 0

    lax.fori_loop(0, npad // L, zero, 0)

    base0 = w * et
    pltpu.sync_copy(dst_hbm.at[pl.ds(base0, et)], dstb)
    pltpu.sync_copy(ew_hbm.at[pl.ds(base0, et)], ewb)

    def group(g, _):
        d16 = dstb[pl.ds(g * L, L)]
        e16 = ewb[pl.ds(g * L, L)]
        plsc.addupdate_scatter(part, [d16], e16)
        return 0

    lax.fori_loop(0, et // L, group, 0)
    pltpu.sync_copy(part, out_hbm.at[w])


def _make_deg(npad, epad):
    et = epad // (NC * NS)
    return pl.kernel(
        functools.partial(_deg_body, npad, et),
        out_type=jax.ShapeDtypeStruct((NC * NS, npad), F32),
        mesh=_mesh(),
        compiler_params=pltpu.CompilerParams(needs_layout_passes=False),
        scratch_types=[
            pltpu.VMEM((npad,), F32),
            pltpu.VMEM((et,), I32),
            pltpu.VMEM((et,), F32),
        ],
    )


SB = 512
CPS = SB // CH


def _agg_body(npad, et, d, feature_split, out_cols, src_hbm, dst_hbm, ew_hbm,
              v_hbm, out_hbm, srcb, dstb, ewb, idxg, idxs, rows, zbuf, acc,
              sg0, sg1, sg2, sg3, ss0, ss1, ss2, ss3):
    c = lax.axis_index("c")
    s = lax.axis_index("s")
    tr = npad // NS
    nvec = d // L
    n_super = et // SB
    z16 = jnp.zeros((L,), F32)
    zi16 = jnp.zeros((L,), I32)
    sg = (sg0, sg1, sg2, sg3)
    ss = (ss0, ss1, ss2, ss3)

    def zb(i, _):
        for k in range(nvec):
            zbuf[i, pl.ds(k * L, L)] = z16
        return 0

    lax.fori_loop(0, ZR, zb, 0)

    row0 = s * tr

    def za(j, _):
        pltpu.sync_copy(zbuf, acc.at[pl.ds(row0 + j * ZR, ZR)])
        return 0

    lax.fori_loop(0, tr // ZR, za, 0)
    plsc.subcore_barrier()

    if feature_split:
        tile_base = s * et
        goff = c * npad
    else:
        tile_base = (c * NS + s) * et
        goff = 0

    def load_edges(sc_i):
        eo = lax.rem(sc_i, 2) * SB
        b = tile_base + sc_i * SB
        pltpu.sync_copy(src_hbm.at[pl.ds(b, SB)], srcb.at[pl.ds(eo, SB)])
        pltpu.sync_copy(dst_hbm.at[pl.ds(b, SB)], dstb.at[pl.ds(eo, SB)])
        pltpu.sync_copy(ew_hbm.at[pl.ds(b, SB)], ewb.at[pl.ds(eo, SB)])

    def build_idx(p, off):
        for g in range(CH // L):
            sl = pl.ds(off + g * L, L)
            idxg[p, pl.ds(g * L, L)] = srcb[sl] + goff
            idxs[p, pl.ds(g * L, L)] = dstb[sl]

    def issue_gather(p):
        pltpu.async_copy(v_hbm.at[idxg.at[p]], rows.at[p], sg[p])

    def wait_gather(p):
        pltpu.make_async_copy(v_hbm.at[idxg.at[p]], rows.at[p], sg[p]).wait()

    def issue_scatter(p):
        pltpu.async_copy(rows.at[p], acc.at[idxs.at[p]], ss[p], add=True)

    def wait_scatter(p):
        pltpu.make_async_copy(rows.at[p], acc.at[idxs.at[p]], ss[p]).wait()

    def scale(p, off):
        def body(g, _):
            e16 = ewb[pl.ds(off + g * L, L)]
            for e in range(L):
                sc = e16[e]
                for k in range(nvec):
                    rows[p, g * L + e, pl.ds(k * L, L)] = (
                        rows[p, g * L + e, pl.ds(k * L, L)] * sc)
            return 0

        lax.fori_loop(0, CH // L, body, 0)

    for q in (2, 3):
        for g in range(CH // L):
            idxs[q, pl.ds(g * L, L)] = zi16

        def zr(i, _):
            for k in range(nvec):
                rows[q, i, pl.ds(k * L, L)] = z16
            return 0

        lax.fori_loop(0, CH, zr, 0)
        issue_scatter(q)

    load_edges(0)
    build_idx(0, 0)
    issue_gather(0)
    build_idx(1, CH)
    issue_gather(1)

    def superchunk(sc_i, _):
        eo_cur = lax.rem(sc_i, 2) * SB
        eo_next = SB - eo_cur
        load_edges(sc_i + 1)
        for j in range(CPS):
            p = j
            q = (j + 2) % 4
            wait_gather(p)
            wait_scatter(q)
            if j < 2:
                build_idx(q, eo_cur + (j + 2) * CH)
            else:
                build_idx(q, eo_next + (j - 2) * CH)
            issue_gather(q)
            scale(p, eo_cur + j * CH)
            issue_scatter(p)
        return 0

    lax.fori_loop(0, n_super, superchunk, 0)
    wait_gather(0)
    wait_gather(1)
    wait_scatter(2)
    wait_scatter(3)

    plsc.subcore_barrier()
    if out_cols == d:
        pltpu.sync_copy(acc.at[pl.ds(row0, tr)],
                        out_hbm.at[pl.ds(c * npad + row0, tr)])
    else:
        pltpu.sync_copy(acc.at[pl.ds(row0, tr), pl.ds(0, out_cols)],
                        out_hbm.at[pl.ds(c * npad + row0, tr)])


def _make_agg(npad, epad, d, feature_split, out_cols):
    et = epad // ((NS if feature_split else NC * NS))
    return pl.kernel(
        functools.partial(_agg_body, npad, et, d, feature_split, out_cols),
        out_type=jax.ShapeDtypeStruct((NC * npad, out_cols), F32),
        mesh=_mesh(),
        compiler_params=pltpu.CompilerParams(
            needs_layout_passes=False, use_tc_tiling_on_sc=False),
        scratch_types=[
            pltpu.VMEM((2 * SB,), I32),
            pltpu.VMEM((2 * SB,), I32),
            pltpu.VMEM((2 * SB,), F32),
            pltpu.VMEM((4, CH), I32),
            pltpu.VMEM((4, CH), I32),
            pltpu.VMEM((4, CH, d), F32),
            pltpu.VMEM((ZR, d), F32),
            pltpu.VMEM_SHARED((npad, d), F32),
            pltpu.SemaphoreType.DMA,
            pltpu.SemaphoreType.DMA,
            pltpu.SemaphoreType.DMA,
            pltpu.SemaphoreType.DMA,
            pltpu.SemaphoreType.DMA,
            pltpu.SemaphoreType.DMA,
            pltpu.SemaphoreType.DMA,
            pltpu.SemaphoreType.DMA,
        ],
    )


def _hid_body(npad, b4, s1_hbm, xs_hbm, dis_hbm, b1_hbm, w2_hbm,
              hs2p_hbm, hs2c_hbm, s0b, s1b, x0b, x1b, disb, b1b, w2b,
              hpb, hcb, smh):
    c = lax.axis_index("c")
    s = lax.axis_index("s")
    w = c * NS + s
    nt = npad // (NC * NS)
    iota = lax.iota(I32, L)
    z16 = jnp.zeros((L,), F32)
    zi16 = jnp.zeros((L,), I32)
    oi16 = jnp.full((L,), 1, I32)

    pltpu.sync_copy(b1_hbm, b1b)
    pltpu.sync_copy(w2_hbm, w2b)

    def zhp(i, _):
        hpb[i, pl.ds(0, L)] = z16
        return 0

    lax.fori_loop(0, b4, zhp, 0)

    base = w * nt

    def blk(bi, _):
        nb0 = base + bi * b4
        d1 = pltpu.async_copy(s1_hbm.at[pl.ds(nb0, b4)],
                              s0b.at[:, pl.ds(0, 32)], smh)
        d2 = pltpu.async_copy(s1_hbm.at[pl.ds(npad + nb0, b4)],
                              s1b.at[:, pl.ds(0, 32)], smh)
        d3 = pltpu.async_copy(xs_hbm.at[pl.ds(nb0, b4)],
                              x0b.at[:, pl.ds(0, 32)], smh)
        d4 = pltpu.async_copy(xs_hbm.at[pl.ds(npad + nb0, b4)],
                              x1b.at[:, pl.ds(0, 32)], smh)
        d5 = pltpu.async_copy(dis_hbm.at[pl.ds(0, 1), pl.ds(nb0, b4)],
                              disb, smh)
        d1.wait()
        d2.wait()
        d3.wait()
        d4.wait()
        d5.wait()

        def grp(g, _):
            rows16 = g * L + iota
            dis16 = disb[0, pl.ds(g * L, L)]
            b1vs = [b1b[pl.ds(q * L, L)] for q in range(4)]
            w2vs = [w2b[pl.ds(q * L, L)] for q in range(8)]
            acc0 = z16
            acc1 = z16
            for k in range(4 * L):
                sb = s0b if k < 2 * L else s1b
                xb = x0b if k < 2 * L else x1b
                colv = jnp.full((L,), k % (2 * L), I32)
                sv = plsc.load_gather(sb, [rows16, colv])
                xv = plsc.load_gather(xb, [rows16, colv])
                h = jnp.maximum(dis16 * (sv + xv) + b1vs[k // L][k % L], 0.0)
                acc0 = acc0 + h * w2vs[(2 * k) // L][(2 * k) % L]
                acc1 = acc1 + h * w2vs[(2 * k + 1) // L][(2 * k + 1) % L]
            o0 = dis16 * acc0
            o1 = dis16 * acc1
            plsc.store_scatter(hpb, [rows16, zi16], o0)
            plsc.store_scatter(hpb, [rows16, oi16], o1)
            plsc.store_scatter(hcb, [rows16, zi16], o0)
            plsc.store_scatter(hcb, [rows16, oi16], o1)
            return 0

        lax.fori_loop(0, b4 // L, grp, 0)
        pltpu.sync_copy(hpb.at[:, pl.ds(0, 16)], hs2p_hbm.at[pl.ds(nb0, b4)])
        pltpu.sync_copy(hcb, hs2c_hbm.at[pl.ds(nb0, b4)])
        return 0

    lax.fori_loop(0, nt // b4, blk, 0)


def _make_hid(npad, d2p, d_out):
    b4 = 224
    return pl.kernel(
        functools.partial(_hid_body, npad, b4),
        out_type=[
            jax.ShapeDtypeStruct((npad, d2p), F32),
            jax.ShapeDtypeStruct((npad, d_out), F32),
        ],
        mesh=_mesh(),
        compiler_params=pltpu.CompilerParams(
            needs_layout_passes=False, use_tc_tiling_on_sc=False),
        scratch_types=[
            pltpu.VMEM((b4, 33), F32),
            pltpu.VMEM((b4, 33), F32),
            pltpu.VMEM((b4, 33), F32),
            pltpu.VMEM((b4, 33), F32),
            pltpu.VMEM((1, b4), F32),
            pltpu.VMEM((64,), F32),
            pltpu.VMEM((128,), F32),
            pltpu.VMEM((b4, 17), F32),
            pltpu.VMEM((b4, d_out), F32),
            pltpu.SemaphoreType.DMA,
        ],
    )


_LOG2_C = (0.04343132, -0.40488876, 1.59397599, -3.4926196, 5.04697861,
           -2.78684575)
_LN2 = 0.6931471805599453


def _out_body(npad, n, b4, dis_hbm, s2_hbm, hc_hbm, b2_hbm, out_hbm,
              s2a, s2b, hcb, disb, b2b, ob, sem):
    c = lax.axis_index("c")
    s = lax.axis_index("s")
    w = c * NS + s
    nt = npad // (NC * NS)
    iota = lax.iota(I32, L)
    zi16 = jnp.zeros((L,), I32)
    oi16 = jnp.full((L,), 1, I32)

    pltpu.sync_copy(b2_hbm, b2b)
    base = w * nt

    def blk(bi, _):
        nb0 = base + bi * b4
        d1 = pltpu.async_copy(s2_hbm.at[pl.ds(nb0, b4)],
                              s2a.at[:, pl.ds(0, 16)], sem)
        d2 = pltpu.async_copy(s2_hbm.at[pl.ds(npad + nb0, b4)],
                              s2b.at[:, pl.ds(0, 16)], sem)
        d3 = pltpu.async_copy(hc_hbm.at[pl.ds(nb0, b4)], hcb, sem)
        d4 = pltpu.async_copy(dis_hbm.at[pl.ds(0, 1), pl.ds(nb0, b4)],
                              disb, sem)
        d1.wait()
        d2.wait()
        d3.wait()
        d4.wait()
        b2v = b2b[pl.ds(0, L)]

        def grp(g, _):
            rows16 = g * L + iota
            dis16 = disb[0, pl.ds(g * L, L)]
            a0 = plsc.load_gather(s2a, [rows16, zi16])
            a1 = plsc.load_gather(s2a, [rows16, oi16])
            c0 = plsc.load_gather(s2b, [rows16, zi16])
            c1 = plsc.load_gather(s2b, [rows16, oi16])
            h0 = plsc.load_gather(hcb, [rows16, zi16])
            h1 = plsc.load_gather(hcb, [rows16, oi16])
            o0 = dis16 * (a0 + c0 + h0) + b2v[0]
            o1 = dis16 * (a1 + c1 + h1) + b2v[1]
            mx = jnp.maximum(o0, o1)
            t = jnp.exp(-jnp.abs(o0 - o1))
            y = 1.0 + t
            bits = plsc.bitcast(y, jnp.int32)
            ev = (lax.shift_right_logical(bits, 23) & 255) - 127
            mant = plsc.bitcast(
                (bits & 0x7FFFFF) | 0x3F800000, F32)
            p = jnp.full((L,), _LOG2_C[0], F32)
            for cc in _LOG2_C[1:]:
                p = p * mant + cc
            lse = mx + _LN2 * (p + ev.astype(F32))
            plsc.store_scatter(ob, [rows16, zi16], o0 - lse)
            plsc.store_scatter(ob, [rows16, oi16], o1 - lse)
            return 0

        lax.fori_loop(0, b4 // L, grp, 0)
        @pl.when(nb0 + b4 <= n)
        def _():
            pltpu.sync_copy(ob, out_hbm.at[pl.ds(nb0, b4)])

        return 0

    lax.fori_loop(0, nt // b4, blk, 0)

    tail_start = (n // b4) * b4
    tail = n - tail_start
    if tail:
        tw = tail_start // nt
        @pl.when(w == tw)
        def _():
            pltpu.sync_copy(ob.at[pl.ds(0, tail)],
                            out_hbm.at[pl.ds(tail_start, tail)])


def _make_out(npad, n, d_out):
    b4 = 224
    return pl.kernel(
        functools.partial(_out_body, npad, n, b4),
        out_type=jax.ShapeDtypeStruct((n, d_out), F32),
        mesh=_mesh(),
        compiler_params=pltpu.CompilerParams(
            needs_layout_passes=False, use_tc_tiling_on_sc=False),
        scratch_types=[
            pltpu.VMEM((b4, 17), F32),
            pltpu.VMEM((b4, 17), F32),
            pltpu.VMEM((b4, d_out), F32),
            pltpu.VMEM((1, b4), F32),
            pltpu.VMEM((L,), F32),
            pltpu.VMEM((b4, d_out), F32),
            pltpu.SemaphoreType.DMA,
        ],
    )


def _k2_body(parts_ref, x_ref, w1_ref, xs_ref, dis1d_ref):
    parts = parts_ref[...]
    nparts = parts.shape[0]
    ones = jnp.ones((nparts, 1), F32)
    deg = 1.0 + lax.dot_general(parts, ones, (((0,), (0,)), ((), ())),
                                preferred_element_type=F32)
    dis = lax.rsqrt(deg)
    deg_row = 1.0 + jnp.dot(jnp.ones((1, nparts), F32), parts,
                            preferred_element_type=F32)
    dis1d_ref[...] = lax.rsqrt(deg_row)
    xw = jnp.dot(x_ref[...], w1_ref[...], preferred_element_type=F32)
    half = xw.shape[1] // 2
    xs_ref[0] = dis * xw[:, :half]
    xs_ref[1] = dis * xw[:, half:]


def kernel(x, edge_index, edge_weight, W1, b1, W2, b2):
    n, d_in = x.shape
    e = edge_index.shape[1]
    d_hid = W1.shape[1]
    d_out = W2.shape[1]
    half = d_hid // 2
    d2p = 16

    npad = -(-n // 512) * 512
    r = 3584 if npad % 3584 == 0 else 512
    nb = npad // r
    epad = -(-e // (NC * NS * CH)) * (NC * NS * CH)

    ei = jnp.pad(edge_index, ((0, 0), (0, epad + SB - e)))
    src = ei[0]
    dst = ei[1]
    ew = jnp.pad(edge_weight, (0, epad + SB - e))
    xp = x

    parts = _make_deg(npad, epad)(dst, ew)

    xs3, dis1d = pl.pallas_call(
        _k2_body,
        grid=(nb,),
        in_specs=[
            pl.BlockSpec((NC * NS, r), lambda j: (0, j)),
            pl.BlockSpec((r, d_in), lambda j: (j, 0)),
            pl.BlockSpec((d_in, d_hid), lambda j: (0, 0)),
        ],
        out_specs=[
            pl.BlockSpec((NC, r, half), lambda j: (0, j, 0)),
            pl.BlockSpec((1, r), lambda j: (0, j)),
        ],
        out_shape=[
            jax.ShapeDtypeStruct((NC, npad, half), F32),
            jax.ShapeDtypeStruct((1, npad), F32),
        ],
    )(parts, xp, W1)
    xs_flat = xs3.reshape(NC * npad, half)

    s1_flat = _make_agg(npad, epad, half, True, half)(src, dst, ew, xs_flat)

    hs2p, hs2c = _make_hid(npad, d2p, d_out)(
        s1_flat, xs_flat, dis1d, b1, W2.reshape(d_hid * d_out))

    s2c = _make_agg(npad, epad, d2p, False, d2p)(src, dst, ew, hs2p)

    return _make_out(npad, n, d_out)(
        dis1d, s2c, hs2c, jnp.pad(b2, (0, L - d_out)))
